# baseline (speedup 1.0000x reference)
import functools

import jax
import jax.numpy as jnp
from jax import lax
from jax.experimental import pallas as pl
from jax.experimental.pallas import tpu as pltpu
from jax.experimental.pallas import tpu_sc as plsc

N_NODES = 10000
N_EDGES = 320000
N_PAIR = 20000
D = 128

NC = 2
NS = 16
NW = NC * NS

NPAD = 10240
ROWS_PER_TILE = NPAD // NS
CHUNK = 128
CPT = 80
EPAD = NW * CPT * CHUNK

GPT0 = 20
GPAD = NW * GPT0 * CHUNK
PPAD = GPAD // 4


def _sage_agg_body(table, src_idx, dst_idx, agg_out, deg_out,
                   src_v, dst_v, rows_v, ones_v, zflat_v,
                   agg_sh, deg_sh, gsem):
    c = lax.axis_index("c")
    s = lax.axis_index("s")
    tile_row0 = s * ROWS_PER_TILE

    def init_zrow(i, _):
        rows_v[i // 8, pl.ds((i % 8) * 16, 16)] = jnp.zeros((16,), jnp.float32)
        return 0
    lax.fori_loop(0, 1024, init_zrow, 0)

    def init_small(i, _):
        zflat_v[pl.ds(i * 16, 16)] = jnp.zeros((16,), jnp.float32)
        return 0
    lax.fori_loop(0, ROWS_PER_TILE // 16, init_small, 0)
    for i in range(8):
        ones_v[pl.ds(i * 16, 16)] = jnp.ones((16,), jnp.float32)

    for b in range(ROWS_PER_TILE // 128):
        pltpu.sync_copy(rows_v, agg_sh.at[pl.ds(tile_row0 + b * 128, 128)])
    pltpu.sync_copy(zflat_v, deg_sh.at[pl.ds(tile_row0, ROWS_PER_TILE)])
    plsc.subcore_barrier()

    wid = s * NC + c
    pltpu.sync_copy(src_idx.at[wid], src_v)
    pltpu.sync_copy(dst_idx.at[wid], dst_v)

    def step(j, _):
        pltpu.async_copy(table.at[src_v.at[j]], rows_v, gsem).wait()
        pltpu.sync_copy(rows_v, agg_sh.at[dst_v.at[j]], add=True)
        pltpu.sync_copy(ones_v, deg_sh.at[dst_v.at[j]], add=True)
        return 0
    lax.fori_loop(0, CPT, step, 0)

    plsc.subcore_barrier()

    for b in range(ROWS_PER_TILE // 128):
        r0 = tile_row0 + b * 128
        pltpu.sync_copy(agg_sh.at[pl.ds(r0, 128)], rows_v)
        pltpu.sync_copy(rows_v, agg_out.at[c].at[pl.ds(r0, 128)])
    pltpu.sync_copy(deg_sh.at[pl.ds(tile_row0, ROWS_PER_TILE)], zflat_v)
    pltpu.sync_copy(zflat_v, deg_out.at[pl.ds(c * NPAD + tile_row0, ROWS_PER_TILE)])


def _sage_agg(table, src2d, dst2d):
    mesh = plsc.VectorSubcoreMesh(core_axis_name="c", subcore_axis_name="s")
    fn = functools.partial(
        pl.kernel,
        mesh=mesh,
        out_type=[
            jax.ShapeDtypeStruct((NC, NPAD, 128), jnp.float32),
            jax.ShapeDtypeStruct((NC * NPAD,), jnp.float32),
        ],
        scratch_types=[
            pltpu.VMEM((CPT, 128), jnp.int32),
            pltpu.VMEM((CPT, 128), jnp.int32),
            pltpu.VMEM((128, 128), jnp.float32),
            pltpu.VMEM((128,), jnp.float32),
            pltpu.VMEM((ROWS_PER_TILE,), jnp.float32),
            pltpu.VMEM_SHARED((NPAD, 128), jnp.float32),
            pltpu.VMEM_SHARED((NPAD,), jnp.float32),
            pltpu.SemaphoreType.DMA,
        ],
    )(_sage_agg_body)
    return fn(table, src2d, dst2d)


def _pair_gather_body(table, idx2d, out, idx_v, rows_v, gsem, osem):
    c = lax.axis_index("c")
    s = lax.axis_index("s")
    wid = s * NC + c
    pltpu.sync_copy(idx2d.at[wid], idx_v)

    def step(j, _):
        pltpu.async_copy(table.at[idx_v.at[j]], rows_v, gsem).wait()
        g = wid * GPT0 + j
        pltpu.async_copy(rows_v, out.at[pl.ds(g * 128, 128)], osem).wait()
        return 0
    lax.fori_loop(0, GPT0, step, 0)


def _pair_gather(table, idx2d):
    mesh = plsc.VectorSubcoreMesh(core_axis_name="c", subcore_axis_name="s")
    fn = functools.partial(
        pl.kernel,
        mesh=mesh,
        out_type=jax.ShapeDtypeStruct((GPAD, 128), jnp.float32),
        scratch_types=[
            pltpu.VMEM((GPT0, 128), jnp.int32),
            pltpu.VMEM((128, 128), jnp.float32),
            pltpu.SemaphoreType.DMA,
            pltpu.SemaphoreType.DMA,
        ],
    )(_pair_gather_body)
    return fn(table, idx2d)


def _layer_tc_body(x_ref, a0_ref, a1_ref, d0_ref, d1_ref,
                   ws_ref, wn_ref, b_ref, o_ref, *, relu):
    deg = d0_ref[...] + d1_ref[...]
    rdeg = (1.0 / jnp.maximum(deg, 1.0)).reshape(1, deg.shape[0])
    rd_mat = lax.dot_general(rdeg, jnp.ones((1, 128), jnp.float32),
                             (((0,), (0,)), ((), ())),
                             preferred_element_type=jnp.float32)
    hn = (a0_ref[0] + a1_ref[0]) * rd_mat
    acc = (jnp.dot(x_ref[...], ws_ref[...], preferred_element_type=jnp.float32)
           + jnp.dot(hn, wn_ref[...], preferred_element_type=jnp.float32)
           + b_ref[...])
    o_ref[...] = jnp.maximum(acc, 0.0) if relu else acc


def _layer_tc(x, aggp, degp, wself_t, wneigh_t, bias, relu):
    BN = 256
    grid = (NPAD // BN,)
    nb = NPAD // BN
    return pl.pallas_call(
        functools.partial(_layer_tc_body, relu=relu),
        grid=grid,
        in_specs=[
            pl.BlockSpec((BN, 128), lambda i: (i, 0)),
            pl.BlockSpec((1, BN, 128), lambda i: (0, i, 0)),
            pl.BlockSpec((1, BN, 128), lambda i: (1, i, 0)),
            pl.BlockSpec((BN,), lambda i: (i,)),
            pl.BlockSpec((BN,), lambda i: (i + nb,)),
            pl.BlockSpec((128, 128), lambda i: (0, 0)),
            pl.BlockSpec((128, 128), lambda i: (0, 0)),
            pl.BlockSpec((1, 128), lambda i: (0, 0)),
        ],
        out_specs=pl.BlockSpec((BN, 128), lambda i: (i, 0)),
        out_shape=jax.ShapeDtypeStruct((NPAD, 128), jnp.float32),
    )(x, aggp, aggp, degp, degp, wself_t, wneigh_t, bias)


def _decoder_tc_body(sp_ref, sn_ref, dp_ref, dn_ref, w1_ref, b1_ref,
                     w2_ref, b2_ref, w3_ref, b3_ref, op_ref, on_ref):
    e = jnp.concatenate([sp_ref[...] * dp_ref[...],
                         sn_ref[...] * dn_ref[...]], axis=0)
    e = jnp.maximum(
        jnp.dot(e, w1_ref[...], preferred_element_type=jnp.float32) + b1_ref[...], 0.0)
    e = jnp.maximum(
        jnp.dot(e, w2_ref[...], preferred_element_type=jnp.float32) + b2_ref[...], 0.0)
    r = jnp.dot(e, w3_ref[...], preferred_element_type=jnp.float32) + b3_ref[...]
    bp = op_ref.shape[0]
    op_ref[...] = r[:bp]
    on_ref[...] = r[bp:]


def _decoder_tc(g, w1t, b1, w2t, b2, w3t, b3):
    BP = 512
    nb = PPAD // BP
    grid = (nb,)
    qspec = lambda q: pl.BlockSpec((BP, 128), lambda i, q=q: (i + q * nb, 0))
    return pl.pallas_call(
        _decoder_tc_body,
        grid=grid,
        in_specs=[
            qspec(0), qspec(1), qspec(2), qspec(3),
            pl.BlockSpec((128, 128), lambda i: (0, 0)),
            pl.BlockSpec((1, 128), lambda i: (0, 0)),
            pl.BlockSpec((128, 128), lambda i: (0, 0)),
            pl.BlockSpec((1, 128), lambda i: (0, 0)),
            pl.BlockSpec((128, 1), lambda i: (0, 0)),
            pl.BlockSpec((1, 1), lambda i: (0, 0)),
        ],
        out_specs=[pl.BlockSpec((BP, 1), lambda i: (i, 0)),
                   pl.BlockSpec((BP, 1), lambda i: (i, 0))],
        out_shape=[jax.ShapeDtypeStruct((PPAD, 1), jnp.float32),
                   jax.ShapeDtypeStruct((PPAD, 1), jnp.float32)],
    )(g, g, g, g, w1t, b1, w2t, b2, w3t, b3)


def _pad_edges(edge_index):
    src = edge_index[0]
    dst = edge_index[1]
    pad = EPAD - N_EDGES
    src = jnp.concatenate([src, jnp.zeros((pad,), jnp.int32)])
    dst = jnp.concatenate([dst, jnp.full((pad,), NPAD - 1, jnp.int32)])
    return src.reshape(NW, CPT, CHUNK), dst.reshape(NW, CPT, CHUNK)


def kernel(x, edge_index_l0, edge_index_l1, pos_edge_index, neg_edge_index,
           Wself0, Wneigh0, b0, Wself1, Wneigh1, b1,
           Wd1, bd1, Wd2, bd2, Wd3, bd3):
    xp = jnp.pad(x, ((0, NPAD - N_NODES), (0, 0)))
    src0, dst0 = _pad_edges(edge_index_l0)
    src1, dst1 = _pad_edges(edge_index_l1)

    aggp0, degp0 = _sage_agg(xp, src0, dst0)
    h0 = _layer_tc(xp, aggp0, degp0, Wself0.T, Wneigh0.T,
                   b0.reshape(1, D), relu=True)

    aggp1, degp1 = _sage_agg(h0, src1, dst1)
    h1 = _layer_tc(h0, aggp1, degp1, Wself1.T, Wneigh1.T,
                   b1.reshape(1, D), relu=False)

    zpad = jnp.zeros((PPAD - N_PAIR,), jnp.int32)
    idx_all = jnp.concatenate([
        pos_edge_index[0], zpad, neg_edge_index[0], zpad,
        pos_edge_index[1], zpad, neg_edge_index[1], zpad,
    ]).reshape(NW, GPT0, CHUNK)
    gathered = _pair_gather(h1, idx_all)

    dec_p, dec_n = _decoder_tc(gathered, Wd1.T, bd1.reshape(1, D), Wd2.T,
                               bd2.reshape(1, D), Wd3.T, bd3.reshape(1, 1))
    return (dec_p[:N_PAIR], dec_n[:N_PAIR])


# speedup vs baseline: 1.3760x; 1.3760x over previous
import functools

import jax
import jax.numpy as jnp
from jax import lax
from jax.experimental import pallas as pl
from jax.experimental.pallas import tpu as pltpu
from jax.experimental.pallas import tpu_sc as plsc

N_NODES = 10000
N_EDGES = 320000
N_PAIR = 20000
D = 128

NC = 2
NS = 16
NW = NC * NS

NPAD = 10240
ROWS_PER_TILE = NPAD // NS
CHUNK = 128
CPT = 79
EPAD = NW * CPT * CHUNK

GPT0 = 20
GPAD = NW * GPT0 * CHUNK
PPAD = GPAD // 4


def _sage_agg_body(table, src_idx, dst_idx, agg_out, deg_out,
                   src_v, dst_v, rows_v, ones_v, zflat_v,
                   agg_sh, deg_sh, gsem):
    c = lax.axis_index("c")
    s = lax.axis_index("s")
    tile_row0 = s * ROWS_PER_TILE

    def init_zrow(i, _):
        rows_v[i // 8, pl.ds((i % 8) * 16, 16)] = jnp.zeros((16,), jnp.float32)
        return 0
    lax.fori_loop(0, 1024, init_zrow, 0)

    def init_small(i, _):
        zflat_v[pl.ds(i * 16, 16)] = jnp.zeros((16,), jnp.float32)
        return 0
    lax.fori_loop(0, ROWS_PER_TILE // 16, init_small, 0)
    for i in range(8):
        ones_v[pl.ds(i * 16, 16)] = jnp.ones((16,), jnp.float32)

    for b in range(ROWS_PER_TILE // 128):
        pltpu.sync_copy(rows_v, agg_sh.at[pl.ds(tile_row0 + b * 128, 128)])
    pltpu.sync_copy(zflat_v, deg_sh.at[pl.ds(tile_row0, ROWS_PER_TILE)])
    plsc.subcore_barrier()

    wid = s * NC + c
    pltpu.sync_copy(src_idx.at[wid], src_v)
    pltpu.sync_copy(dst_idx.at[wid], dst_v)

    def step(j, _):
        pltpu.async_copy(table.at[src_v.at[j]], rows_v, gsem).wait()
        pltpu.sync_copy(rows_v, agg_sh.at[dst_v.at[j]], add=True)
        pltpu.sync_copy(ones_v, deg_sh.at[dst_v.at[j]], add=True)
        return 0
    lax.fori_loop(0, CPT, step, 0)

    plsc.subcore_barrier()

    for b in range(ROWS_PER_TILE // 128):
        r0 = tile_row0 + b * 128
        pltpu.sync_copy(agg_sh.at[pl.ds(r0, 128)], rows_v)
        pltpu.sync_copy(rows_v, agg_out.at[c].at[pl.ds(r0, 128)])
    pltpu.sync_copy(deg_sh.at[pl.ds(tile_row0, ROWS_PER_TILE)], zflat_v)
    pltpu.sync_copy(zflat_v, deg_out.at[pl.ds(c * NPAD + tile_row0, ROWS_PER_TILE)])


def _sage_agg(table, src2d, dst2d):
    mesh = plsc.VectorSubcoreMesh(core_axis_name="c", subcore_axis_name="s")
    fn = functools.partial(
        pl.kernel,
        mesh=mesh,
        out_type=[
            jax.ShapeDtypeStruct((NC, NPAD, 128), jnp.float32),
            jax.ShapeDtypeStruct((NC * NPAD,), jnp.float32),
        ],
        scratch_types=[
            pltpu.VMEM((CPT, 128), jnp.int32),
            pltpu.VMEM((CPT, 128), jnp.int32),
            pltpu.VMEM((128, 128), jnp.float32),
            pltpu.VMEM((128,), jnp.float32),
            pltpu.VMEM((ROWS_PER_TILE,), jnp.float32),
            pltpu.VMEM_SHARED((NPAD, 128), jnp.float32),
            pltpu.VMEM_SHARED((NPAD,), jnp.float32),
            pltpu.SemaphoreType.DMA,
        ],
    )(_sage_agg_body)
    return fn(table, src2d, dst2d)


def _pair_gather_body(table, idx2d, out, idx_v, rows_v, gsem, osem):
    c = lax.axis_index("c")
    s = lax.axis_index("s")
    wid = s * NC + c
    pltpu.sync_copy(idx2d.at[wid], idx_v)

    def step(j, _):
        pltpu.async_copy(table.at[idx_v.at[j]], rows_v, gsem).wait()
        g = wid * GPT0 + j
        pltpu.async_copy(rows_v, out.at[pl.ds(g * 128, 128)], osem).wait()
        return 0
    lax.fori_loop(0, GPT0, step, 0)


def _pair_gather(table, idx2d):
    mesh = plsc.VectorSubcoreMesh(core_axis_name="c", subcore_axis_name="s")
    fn = functools.partial(
        pl.kernel,
        mesh=mesh,
        out_type=jax.ShapeDtypeStruct((GPAD, 128), jnp.float32),
        scratch_types=[
            pltpu.VMEM((GPT0, 128), jnp.int32),
            pltpu.VMEM((128, 128), jnp.float32),
            pltpu.SemaphoreType.DMA,
            pltpu.SemaphoreType.DMA,
        ],
    )(_pair_gather_body)
    return fn(table, idx2d)


def _layer_tc_body(x_ref, a0_ref, a1_ref, d0_ref, d1_ref,
                   ws_ref, wn_ref, b_ref, o_ref, *, relu):
    deg = d0_ref[...] + d1_ref[...]
    rdeg = (1.0 / jnp.maximum(deg, 1.0)).reshape(1, deg.shape[0])
    rd_mat = lax.dot_general(rdeg, jnp.ones((1, 128), jnp.float32),
                             (((0,), (0,)), ((), ())),
                             preferred_element_type=jnp.float32)
    hn = (a0_ref[0] + a1_ref[0]) * rd_mat
    acc = (jnp.dot(x_ref[...], ws_ref[...], preferred_element_type=jnp.float32)
           + jnp.dot(hn, wn_ref[...], preferred_element_type=jnp.float32)
           + b_ref[...])
    o_ref[...] = jnp.maximum(acc, 0.0) if relu else acc


def _layer_tc(x, aggp, degp, wself_t, wneigh_t, bias, relu):
    BN = 256
    grid = (NPAD // BN,)
    nb = NPAD // BN
    return pl.pallas_call(
        functools.partial(_layer_tc_body, relu=relu),
        grid=grid,
        in_specs=[
            pl.BlockSpec((BN, 128), lambda i: (i, 0)),
            pl.BlockSpec((1, BN, 128), lambda i: (0, i, 0)),
            pl.BlockSpec((1, BN, 128), lambda i: (1, i, 0)),
            pl.BlockSpec((BN,), lambda i: (i,)),
            pl.BlockSpec((BN,), lambda i: (i + nb,)),
            pl.BlockSpec((128, 128), lambda i: (0, 0)),
            pl.BlockSpec((128, 128), lambda i: (0, 0)),
            pl.BlockSpec((1, 128), lambda i: (0, 0)),
        ],
        out_specs=pl.BlockSpec((BN, 128), lambda i: (i, 0)),
        out_shape=jax.ShapeDtypeStruct((NPAD, 128), jnp.float32),
    )(x, aggp, aggp, degp, degp, wself_t, wneigh_t, bias)


def _decoder_tc_body(sp_ref, sn_ref, dp_ref, dn_ref, w1_ref, b1_ref,
                     w2_ref, b2_ref, w3_ref, b3_ref, op_ref, on_ref):
    e = jnp.concatenate([sp_ref[...] * dp_ref[...],
                         sn_ref[...] * dn_ref[...]], axis=0)
    e = jnp.maximum(
        jnp.dot(e, w1_ref[...], preferred_element_type=jnp.float32) + b1_ref[...], 0.0)
    e = jnp.maximum(
        jnp.dot(e, w2_ref[...], preferred_element_type=jnp.float32) + b2_ref[...], 0.0)
    r = jnp.dot(e, w3_ref[...], preferred_element_type=jnp.float32) + b3_ref[...]
    bp = op_ref.shape[0]
    op_ref[...] = r[:bp]
    on_ref[...] = r[bp:]


def _decoder_tc(g, w1t, b1, w2t, b2, w3t, b3):
    BP = 512
    nb = PPAD // BP
    grid = (nb,)
    qspec = lambda q: pl.BlockSpec((BP, 128), lambda i, q=q: (i + q * nb, 0))
    return pl.pallas_call(
        _decoder_tc_body,
        grid=grid,
        in_specs=[
            qspec(0), qspec(1), qspec(2), qspec(3),
            pl.BlockSpec((128, 128), lambda i: (0, 0)),
            pl.BlockSpec((1, 128), lambda i: (0, 0)),
            pl.BlockSpec((128, 128), lambda i: (0, 0)),
            pl.BlockSpec((1, 128), lambda i: (0, 0)),
            pl.BlockSpec((128, 1), lambda i: (0, 0)),
            pl.BlockSpec((1, 1), lambda i: (0, 0)),
        ],
        out_specs=[pl.BlockSpec((BP, 1), lambda i: (i, 0)),
                   pl.BlockSpec((BP, 1), lambda i: (i, 0))],
        out_shape=[jax.ShapeDtypeStruct((PPAD, 1), jnp.float32),
                   jax.ShapeDtypeStruct((PPAD, 1), jnp.float32)],
    )(g, g, g, g, w1t, b1, w2t, b2, w3t, b3)


def _pad_edges(edge_index):
    src = edge_index[0]
    dst = edge_index[1]
    pad = EPAD - N_EDGES
    src = jnp.concatenate([src, jnp.zeros((pad,), jnp.int32)])
    trash = N_NODES + (jnp.arange(pad, dtype=jnp.int32) % (NPAD - N_NODES))
    dst = jnp.concatenate([dst, trash])
    return src.reshape(NW, CPT, CHUNK), dst.reshape(NW, CPT, CHUNK)


def kernel(x, edge_index_l0, edge_index_l1, pos_edge_index, neg_edge_index,
           Wself0, Wneigh0, b0, Wself1, Wneigh1, b1,
           Wd1, bd1, Wd2, bd2, Wd3, bd3):
    xp = jnp.pad(x, ((0, NPAD - N_NODES), (0, 0)))
    src0, dst0 = _pad_edges(edge_index_l0)
    src1, dst1 = _pad_edges(edge_index_l1)

    aggp0, degp0 = _sage_agg(xp, src0, dst0)
    h0 = _layer_tc(xp, aggp0, degp0, Wself0.T, Wneigh0.T,
                   b0.reshape(1, D), relu=True)

    aggp1, degp1 = _sage_agg(h0, src1, dst1)
    h1 = _layer_tc(h0, aggp1, degp1, Wself1.T, Wneigh1.T,
                   b1.reshape(1, D), relu=False)

    zpad = jnp.zeros((PPAD - N_PAIR,), jnp.int32)
    idx_all = jnp.concatenate([
        pos_edge_index[0], zpad, neg_edge_index[0], zpad,
        pos_edge_index[1], zpad, neg_edge_index[1], zpad,
    ]).reshape(NW, GPT0, CHUNK)
    gathered = _pair_gather(h1, idx_all)

    dec_p, dec_n = _decoder_tc(gathered, Wd1.T, bd1.reshape(1, D), Wd2.T,
                               bd2.reshape(1, D), Wd3.T, bd3.reshape(1, 1))
    return (dec_p[:N_PAIR], dec_n[:N_PAIR])
